# Initial kernel scaffold; baseline (speedup 1.0000x reference)
import jax
import jax.numpy as jnp
from jax.experimental import pallas as pl


def kernel(x, edge_index, W1, b1, W2, b2):
    raise NotImplementedError("write your pallas kernel here")


# speedup vs baseline: 9.2176x; 9.2176x over previous
import jax
import jax.numpy as jnp
from jax import lax
from jax.experimental import pallas as pl
from jax.experimental.pallas import tpu as pltpu
from jax.experimental.pallas import tpu_sc as plsc

N = 10000
E = 320000
F_IN = 128
HID = 128
CLS = 40
CPAD = 48

NC, NS = 2, 16
NW = NC * NS
NPAD = 10112
RPT = NPAD // NS
CHUNK = 128
EPT = 10240
NCHUNK = EPT // CHUNK
DEG_W = 16
RB = 128


def _mesh():
    return plsc.VectorSubcoreMesh(
        core_axis_name="c", subcore_axis_name="s",
        num_cores=NC, num_subcores=NS)


_SC_PARAMS = pltpu.CompilerParams(use_tc_tiling_on_sc=False)


def _deg_body(col_hbm, ones_hbm, zeros_hbm, out_hbm, col_v, ones_v, acc):
    c = lax.axis_index("c")
    s = lax.axis_index("s")
    wid = s * NC + c
    pltpu.sync_copy(zeros_hbm.at[pl.ds(s * RPT, RPT)],
                    acc.at[pl.ds(s * RPT, RPT)])
    pltpu.sync_copy(col_hbm.at[wid], col_v)
    pltpu.sync_copy(ones_hbm, ones_v)
    plsc.subcore_barrier()

    def body(j, carry):
        pltpu.sync_copy(ones_v, acc.at[col_v.at[j]], add=True)
        return carry

    lax.fori_loop(0, NCHUNK, body, 0)
    plsc.subcore_barrier()
    pltpu.sync_copy(acc.at[pl.ds(s * RPT, RPT)],
                    out_hbm.at[c, pl.ds(s * RPT, RPT)])


def _sc_degree(col_tiles, ones, zeros16):
    return pl.kernel(
        _deg_body,
        out_type=jax.ShapeDtypeStruct((NC, NPAD, DEG_W), jnp.float32),
        mesh=_mesh(),
        scratch_types=[
            pltpu.VMEM((NCHUNK, CHUNK), jnp.int32),
            pltpu.VMEM((CHUNK, DEG_W), jnp.float32),
            pltpu.VMEM_SHARED((NPAD, DEG_W), jnp.float32),
        ],
        compiler_params=_SC_PARAMS,
    )(col_tiles, ones, zeros16)


def _prop_body(y_hbm, row_hbm, col_hbm, zeros_hbm, out_hbm,
               row_v, col_v, buf, acc):
    c = lax.axis_index("c")
    s = lax.axis_index("s")
    wid = s * NC + c
    pltpu.sync_copy(zeros_hbm.at[pl.ds(s * RPT, RPT)],
                    acc.at[pl.ds(s * RPT, RPT)])
    pltpu.sync_copy(row_hbm.at[wid], row_v)
    pltpu.sync_copy(col_hbm.at[wid], col_v)
    plsc.subcore_barrier()

    def body(j, carry):
        pltpu.sync_copy(y_hbm.at[row_v.at[j]], buf)
        pltpu.sync_copy(buf, acc.at[col_v.at[j]], add=True)
        return carry

    lax.fori_loop(0, NCHUNK, body, 0)
    plsc.subcore_barrier()
    pltpu.sync_copy(acc.at[pl.ds(s * RPT, RPT)],
                    out_hbm.at[c, pl.ds(s * RPT, RPT)])


def _sc_propagate(y, row_tiles, col_tiles, zeros, width):
    return pl.kernel(
        _prop_body,
        out_type=jax.ShapeDtypeStruct((NC, NPAD, width), jnp.float32),
        mesh=_mesh(),
        scratch_types=[
            pltpu.VMEM((NCHUNK, CHUNK), jnp.int32),
            pltpu.VMEM((NCHUNK, CHUNK), jnp.int32),
            pltpu.VMEM((CHUNK, width), jnp.float32),
            pltpu.VMEM_SHARED((NPAD, width), jnp.float32),
        ],
        compiler_params=_SC_PARAMS,
    )(y, row_tiles, col_tiles, zeros)


def _dinv(degp_ref):
    deg = degp_ref[0, :, 0:1] + degp_ref[1, :, 0:1] + 1.0
    return lax.rsqrt(deg)


def _lin1_body(x_ref, w_ref, degp_ref, y_ref):
    y_ref[...] = jnp.dot(x_ref[...], w_ref[...],
                         preferred_element_type=jnp.float32) * _dinv(degp_ref)


def _tc_lin1(xpad, W1, degp):
    grid = (NPAD // RB,)
    return pl.pallas_call(
        _lin1_body,
        grid=grid,
        in_specs=[
            pl.BlockSpec((RB, F_IN), lambda i: (i, 0)),
            pl.BlockSpec((F_IN, HID), lambda i: (0, 0)),
            pl.BlockSpec((NC, RB, DEG_W), lambda i: (0, i, 0)),
        ],
        out_specs=pl.BlockSpec((RB, HID), lambda i: (i, 0)),
        out_shape=jax.ShapeDtypeStruct((NPAD, HID), jnp.float32),
    )(xpad, W1, degp)


def _lin2_body(sp_ref, y1_ref, degp_ref, b1_ref, w2_ref, y2_ref):
    dinv = _dinv(degp_ref)
    pre = dinv * (sp_ref[0] + sp_ref[1] + y1_ref[...]) + b1_ref[...]
    h = jnp.maximum(pre, 0.0)
    y2_ref[...] = jnp.dot(h, w2_ref[...],
                          preferred_element_type=jnp.float32) * dinv


def _tc_lin2(s1, y1, degp, b1r, W2p):
    grid = (NPAD // RB,)
    return pl.pallas_call(
        _lin2_body,
        grid=grid,
        in_specs=[
            pl.BlockSpec((NC, RB, HID), lambda i: (0, i, 0)),
            pl.BlockSpec((RB, HID), lambda i: (i, 0)),
            pl.BlockSpec((NC, RB, DEG_W), lambda i: (0, i, 0)),
            pl.BlockSpec((1, HID), lambda i: (0, 0)),
            pl.BlockSpec((HID, CPAD), lambda i: (0, 0)),
        ],
        out_specs=pl.BlockSpec((RB, CPAD), lambda i: (i, 0)),
        out_shape=jax.ShapeDtypeStruct((NPAD, CPAD), jnp.float32),
    )(s1, y1, degp, b1r, W2p)


def _out_body(tp_ref, y2_ref, degp_ref, b2_ref, o_ref):
    dinv = _dinv(degp_ref)
    z = dinv * (tp_ref[0] + tp_ref[1] + y2_ref[...]) + b2_ref[...]
    colid = lax.broadcasted_iota(jnp.int32, z.shape, 1)
    z = jnp.where(colid < CLS, z, -1e30)
    m = jnp.max(z, axis=1, keepdims=True)
    lse = jnp.log(jnp.sum(jnp.exp(z - m), axis=1, keepdims=True)) + m
    o_ref[...] = z - lse


def _tc_out(t1, y2, degp, b2p):
    grid = (NPAD // RB,)
    return pl.pallas_call(
        _out_body,
        grid=grid,
        in_specs=[
            pl.BlockSpec((NC, RB, CPAD), lambda i: (0, i, 0)),
            pl.BlockSpec((RB, CPAD), lambda i: (i, 0)),
            pl.BlockSpec((NC, RB, DEG_W), lambda i: (0, i, 0)),
            pl.BlockSpec((1, CPAD), lambda i: (0, 0)),
        ],
        out_specs=pl.BlockSpec((RB, CPAD), lambda i: (i, 0)),
        out_shape=jax.ShapeDtypeStruct((NPAD, CPAD), jnp.float32),
    )(t1, y2, degp, b2p)


def kernel(x, edge_index, W1, b1, W2, b2):
    ei = edge_index.astype(jnp.int32)
    padn = NW * EPT - E
    rowp = jnp.concatenate(
        [ei[0], jnp.full((padn,), N, jnp.int32)]).reshape(NW, NCHUNK, CHUNK)
    colp = jnp.concatenate(
        [ei[1], jnp.full((padn,), N, jnp.int32)]).reshape(NW, NCHUNK, CHUNK)
    xpad = jnp.zeros((NPAD, F_IN), jnp.float32).at[:N].set(x)
    ones16 = jnp.ones((CHUNK, DEG_W), jnp.float32)
    z16 = jnp.zeros((NPAD, DEG_W), jnp.float32)
    z128 = jnp.zeros((NPAD, HID), jnp.float32)
    z48 = jnp.zeros((NPAD, CPAD), jnp.float32)
    W2p = jnp.zeros((HID, CPAD), jnp.float32).at[:, :CLS].set(W2)
    b2p = jnp.zeros((1, CPAD), jnp.float32).at[0, :CLS].set(b2)
    b1r = b1.reshape(1, HID)

    degp = _sc_degree(colp, ones16, z16)
    y1 = _tc_lin1(xpad, W1, degp)
    s1 = _sc_propagate(y1, rowp, colp, z128, HID)
    y2 = _tc_lin2(s1, y1, degp, b1r, W2p)
    t1 = _sc_propagate(y2, rowp, colp, z48, CPAD)
    out = _tc_out(t1, y2, degp, b2p)
    return out[:N, :CLS]


# speedup vs baseline: 10.4137x; 1.1298x over previous
import jax
import jax.numpy as jnp
from jax import lax
from jax.experimental import pallas as pl
from jax.experimental.pallas import tpu as pltpu
from jax.experimental.pallas import tpu_sc as plsc

N = 10000
E = 320000
F_IN = 128
HID = 128
CLS = 40
CPAD = 48

NC, NS = 2, 16
NW = NC * NS
NPAD = 10112
RPT = NPAD // NS
CHUNK = 128
EPT = 10240
NCHUNK = EPT // CHUNK
DEG_W = 16
RB = 128


def _mesh():
    return plsc.VectorSubcoreMesh(
        core_axis_name="c", subcore_axis_name="s",
        num_cores=NC, num_subcores=NS)


_SC_PARAMS = pltpu.CompilerParams(use_tc_tiling_on_sc=False)


def _deg_body(col_hbm, ones_hbm, zeros_hbm, out_hbm, col_v, ones_v, acc):
    c = lax.axis_index("c")
    s = lax.axis_index("s")
    wid = s * NC + c
    pltpu.sync_copy(zeros_hbm.at[pl.ds(s * RPT, RPT)],
                    acc.at[pl.ds(s * RPT, RPT)])
    pltpu.sync_copy(col_hbm.at[wid], col_v)
    pltpu.sync_copy(ones_hbm, ones_v)
    plsc.subcore_barrier()

    def body(j, carry):
        pltpu.sync_copy(ones_v, acc.at[col_v.at[j]], add=True)
        return carry

    lax.fori_loop(0, NCHUNK, body, 0)
    plsc.subcore_barrier()
    pltpu.sync_copy(acc.at[pl.ds(s * RPT, RPT)],
                    out_hbm.at[c, pl.ds(s * RPT, RPT)])


def _sc_degree(col_tiles, ones, zeros16):
    return pl.kernel(
        _deg_body,
        out_type=jax.ShapeDtypeStruct((NC, NPAD, DEG_W), jnp.float32),
        mesh=_mesh(),
        scratch_types=[
            pltpu.VMEM((NCHUNK, CHUNK), jnp.int32),
            pltpu.VMEM((CHUNK, DEG_W), jnp.float32),
            pltpu.VMEM_SHARED((NPAD, DEG_W), jnp.float32),
        ],
        compiler_params=_SC_PARAMS,
    )(col_tiles, ones, zeros16)


NBUF = 8


def _prop_pass(y2d, col_v, out2d, bufs, acc, gsems, ssems, row_v, s, c):
    for b in range(NBUF - 1):
        pltpu.async_copy(y2d.at[row_v.at[b]], bufs[b], gsems[b])

    def body(j2, carry):
        for b in range(NBUF):
            j = j2 * NBUF + b
            nb = (b + NBUF - 1) % NBUF
            pltpu.make_async_copy(y2d.at[row_v.at[j]], bufs[b],
                                  gsems[b]).wait()
            pltpu.async_copy(bufs[b], acc.at[col_v.at[j]], ssems[b], add=True)
            def _wait_prev(j=j, nb=nb):
                pltpu.make_async_copy(bufs[nb], acc.at[col_v.at[j]],
                                      ssems[nb]).wait()
            if b == 0:
                pl.when(j2 > 0)(_wait_prev)
            else:
                _wait_prev()
            jn = jnp.minimum(j + NBUF - 1, NCHUNK - 1)
            pltpu.async_copy(y2d.at[row_v.at[jn]], bufs[nb], gsems[nb])
        return carry

    lax.fori_loop(0, NCHUNK // NBUF, body, 0)
    lb = (NCHUNK - 1) % NBUF
    pltpu.make_async_copy(bufs[lb], acc.at[col_v.at[0]], ssems[lb]).wait()
    for b in range(NBUF - 1):
        pltpu.make_async_copy(y2d.at[row_v.at[0]], bufs[b], gsems[b]).wait()
    plsc.subcore_barrier()
    pltpu.sync_copy(acc.at[pl.ds(s * RPT, RPT)],
                    out2d.at[pl.ds(s * RPT, RPT)])


def _make_prop_body(npass):
    def body(y_hbm, row_hbm, col_hbm, zeros_hbm, out_hbm,
             row_v, col_v, b0, b1, b2, b3, b4, b5, b6, b7, acc,
             g0, g1, g2, g3, g4, g5, g6, g7,
             s0, s1, s2, s3, s4, s5, s6, s7):
        bufs = (b0, b1, b2, b3, b4, b5, b6, b7)
        gsems = (g0, g1, g2, g3, g4, g5, g6, g7)
        ssems = (s0, s1, s2, s3, s4, s5, s6, s7)
        c = lax.axis_index("c")
        s = lax.axis_index("s")
        wid = s * NC + c
        pltpu.sync_copy(row_hbm.at[wid], row_v)
        pltpu.sync_copy(col_hbm.at[wid], col_v)
        for p in range(npass):
            pltpu.sync_copy(zeros_hbm.at[pl.ds(s * RPT, RPT)],
                            acc.at[pl.ds(s * RPT, RPT)])
            plsc.subcore_barrier()
            _prop_pass(y_hbm.at[p], col_v, out_hbm.at[c, p], bufs, acc,
                       gsems, ssems, row_v, s, c)
    return body


def _sc_propagate(y, row_tiles, col_tiles, zeros, width, npass):
    return pl.kernel(
        _make_prop_body(npass),
        out_type=jax.ShapeDtypeStruct((NC, npass, NPAD, width), jnp.float32),
        mesh=_mesh(),
        scratch_types=[
            pltpu.VMEM((NCHUNK, CHUNK), jnp.int32),
            pltpu.VMEM((NCHUNK, CHUNK), jnp.int32),
        ] + [pltpu.VMEM((CHUNK, width), jnp.float32)] * NBUF + [
            pltpu.VMEM_SHARED((NPAD, width), jnp.float32),
        ] + [pltpu.SemaphoreType.DMA] * (2 * NBUF),
        compiler_params=_SC_PARAMS,
    )(y, row_tiles, col_tiles, zeros)


def _dinv(degp_ref):
    deg = degp_ref[0, :, 0:1] + degp_ref[1, :, 0:1] + 1.0
    return lax.rsqrt(deg)


def _lin1_body(x_ref, w_ref, degp_ref, y_ref):
    y = jnp.dot(x_ref[...], w_ref[...],
                preferred_element_type=jnp.float32) * _dinv(degp_ref)
    y_ref[0] = y[:, :HID // 2]
    y_ref[1] = y[:, HID // 2:]


def _tc_lin1(xpad, W1, degp):
    grid = (NPAD // RB,)
    return pl.pallas_call(
        _lin1_body,
        grid=grid,
        in_specs=[
            pl.BlockSpec((RB, F_IN), lambda i: (i, 0)),
            pl.BlockSpec((F_IN, HID), lambda i: (0, 0)),
            pl.BlockSpec((NC, RB, DEG_W), lambda i: (0, i, 0)),
        ],
        out_specs=pl.BlockSpec((2, RB, HID // 2), lambda i: (0, i, 0)),
        out_shape=jax.ShapeDtypeStruct((2, NPAD, HID // 2), jnp.float32),
    )(xpad, W1, degp)


def _lin2_body(sp_ref, y1_ref, degp_ref, b1_ref, w2_ref, y2_ref):
    dinv = _dinv(degp_ref)
    tot = sp_ref[0] + sp_ref[1] + y1_ref[...]
    pre = dinv * jnp.concatenate([tot[0], tot[1]], axis=1) + b1_ref[...]
    h = jnp.maximum(pre, 0.0)
    y2_ref[...] = jnp.dot(h, w2_ref[...],
                          preferred_element_type=jnp.float32) * dinv


def _tc_lin2(s1, y1, degp, b1r, W2p):
    grid = (NPAD // RB,)
    return pl.pallas_call(
        _lin2_body,
        grid=grid,
        in_specs=[
            pl.BlockSpec((NC, 2, RB, HID // 2), lambda i: (0, 0, i, 0)),
            pl.BlockSpec((2, RB, HID // 2), lambda i: (0, i, 0)),
            pl.BlockSpec((NC, RB, DEG_W), lambda i: (0, i, 0)),
            pl.BlockSpec((1, HID), lambda i: (0, 0)),
            pl.BlockSpec((HID, CPAD), lambda i: (0, 0)),
        ],
        out_specs=pl.BlockSpec((RB, CPAD), lambda i: (i, 0)),
        out_shape=jax.ShapeDtypeStruct((NPAD, CPAD), jnp.float32),
    )(s1, y1, degp, b1r, W2p)


def _out_body(tp_ref, y2_ref, degp_ref, b2_ref, o_ref):
    dinv = _dinv(degp_ref)
    z = dinv * (tp_ref[0] + tp_ref[1] + y2_ref[...]) + b2_ref[...]
    colid = lax.broadcasted_iota(jnp.int32, z.shape, 1)
    z = jnp.where(colid < CLS, z, -1e30)
    m = jnp.max(z, axis=1, keepdims=True)
    lse = jnp.log(jnp.sum(jnp.exp(z - m), axis=1, keepdims=True)) + m
    o_ref[...] = z - lse


def _tc_out(t1, y2, degp, b2p):
    grid = (NPAD // RB,)
    return pl.pallas_call(
        _out_body,
        grid=grid,
        in_specs=[
            pl.BlockSpec((NC, RB, CPAD), lambda i: (0, i, 0)),
            pl.BlockSpec((RB, CPAD), lambda i: (i, 0)),
            pl.BlockSpec((NC, RB, DEG_W), lambda i: (0, i, 0)),
            pl.BlockSpec((1, CPAD), lambda i: (0, 0)),
        ],
        out_specs=pl.BlockSpec((RB, CPAD), lambda i: (i, 0)),
        out_shape=jax.ShapeDtypeStruct((NPAD, CPAD), jnp.float32),
    )(t1, y2, degp, b2p)


def kernel(x, edge_index, W1, b1, W2, b2):
    ei = edge_index.astype(jnp.int32)
    padn = NW * EPT - E
    rowp = jnp.concatenate(
        [ei[0], jnp.full((padn,), N, jnp.int32)]).reshape(NW, NCHUNK, CHUNK)
    colp = jnp.concatenate(
        [ei[1], jnp.full((padn,), N, jnp.int32)]).reshape(NW, NCHUNK, CHUNK)
    xpad = jnp.zeros((NPAD, F_IN), jnp.float32).at[:N].set(x)
    ones16 = jnp.ones((CHUNK, DEG_W), jnp.float32)
    z16 = jnp.zeros((NPAD, DEG_W), jnp.float32)
    z64 = jnp.zeros((NPAD, HID // 2), jnp.float32)
    z48 = jnp.zeros((NPAD, CPAD), jnp.float32)
    W2p = jnp.zeros((HID, CPAD), jnp.float32).at[:, :CLS].set(W2)
    b2p = jnp.zeros((1, CPAD), jnp.float32).at[0, :CLS].set(b2)
    b1r = b1.reshape(1, HID)

    degp = _sc_degree(colp, ones16, z16)
    y1 = _tc_lin1(xpad, W1, degp)
    s1 = _sc_propagate(y1, rowp, colp, z64, HID // 2, 2)
    y2 = _tc_lin2(s1, y1, degp, b1r, W2p)
    t1 = _sc_propagate(y2.reshape(1, NPAD, CPAD), rowp, colp, z48, CPAD, 1)
    out = _tc_out(t1.reshape(NC, NPAD, CPAD), y2, degp, b2p)
    return out[:N, :CLS]


# speedup vs baseline: 21.5949x; 2.0737x over previous
import jax
import jax.numpy as jnp
from jax import lax
from jax.experimental import pallas as pl
from jax.experimental.pallas import tpu as pltpu
from jax.experimental.pallas import tpu_sc as plsc

N = 10000
E = 320000
F_IN = 128
HID = 128
CLS = 40
CPAD = 48

NC, NS = 2, 16
NW = NC * NS
NPAD = 10112
RPT = NPAD // NS
CHUNK = 80
EPT = 10240
NCHUNK = EPT // CHUNK
DEG_W = 16
RB = 128


def _mesh():
    return plsc.VectorSubcoreMesh(
        core_axis_name="c", subcore_axis_name="s",
        num_cores=NC, num_subcores=NS)


_SC_PARAMS = pltpu.CompilerParams(use_tc_tiling_on_sc=False)


def _deg_body(col_hbm, ones_hbm, zeros_hbm, out_hbm, col_v, ones_v, acc):
    c = lax.axis_index("c")
    s = lax.axis_index("s")
    wid = s * NC + c
    pltpu.sync_copy(zeros_hbm.at[pl.ds(s * RPT, RPT)],
                    acc.at[pl.ds(s * RPT, RPT)])
    pltpu.sync_copy(col_hbm.at[wid], col_v)
    pltpu.sync_copy(ones_hbm, ones_v)
    plsc.subcore_barrier()

    def body(j, carry):
        pltpu.sync_copy(ones_v, acc.at[col_v.at[j]], add=True)
        return carry

    lax.fori_loop(0, NCHUNK, body, 0)
    plsc.subcore_barrier()
    pltpu.sync_copy(acc.at[pl.ds(s * RPT, RPT)],
                    out_hbm.at[c, pl.ds(s * RPT, RPT)])


def _sc_degree(col_tiles, ones, zeros16):
    return pl.kernel(
        _deg_body,
        out_type=jax.ShapeDtypeStruct((NC, NPAD, DEG_W), jnp.float32),
        mesh=_mesh(),
        scratch_types=[
            pltpu.VMEM((NCHUNK, CHUNK), jnp.int32),
            pltpu.VMEM((CHUNK, DEG_W), jnp.float32),
            pltpu.VMEM_SHARED((NPAD, DEG_W), jnp.float32),
        ],
        compiler_params=_SC_PARAMS,
    )(col_tiles, ones, zeros16)


NBUF = 4


def _prop_pass(y2d, col_v, out2d, bufs, acc, gsems, ssems, row_v, s, c):
    for b in range(NBUF - 1):
        pltpu.async_copy(y2d.at[row_v.at[b]], bufs[b], gsems[b])

    def body(j2, carry):
        for b in range(NBUF):
            j = j2 * NBUF + b
            nb = (b + NBUF - 1) % NBUF
            pltpu.make_async_copy(y2d.at[row_v.at[j]], bufs[b],
                                  gsems[b]).wait()
            pltpu.async_copy(bufs[b], acc.at[col_v.at[j]], ssems[b], add=True)
            def _wait_prev(j=j, nb=nb):
                pltpu.make_async_copy(bufs[nb], acc.at[col_v.at[j]],
                                      ssems[nb]).wait()
            if b == 0:
                pl.when(j2 > 0)(_wait_prev)
            else:
                _wait_prev()
            jn = jnp.minimum(j + NBUF - 1, NCHUNK - 1)
            pltpu.async_copy(y2d.at[row_v.at[jn]], bufs[nb], gsems[nb])
        return carry

    lax.fori_loop(0, NCHUNK // NBUF, body, 0)
    lb = (NCHUNK - 1) % NBUF
    pltpu.make_async_copy(bufs[lb], acc.at[col_v.at[0]], ssems[lb]).wait()
    for b in range(NBUF - 1):
        pltpu.make_async_copy(y2d.at[row_v.at[0]], bufs[b], gsems[b]).wait()
    plsc.subcore_barrier()
    pltpu.sync_copy(acc.at[pl.ds(s * RPT, RPT)],
                    out2d.at[pl.ds(s * RPT, RPT)])


def _make_prop_body(npass):
    def body(y_hbm, row_hbm, col_hbm, zeros_hbm, out_hbm,
             row_v, col_v, b0, b1, b2, b3, ystage, acc,
             g0, g1, g2, g3, s0, s1, s2, s3):
        bufs = (b0, b1, b2, b3)
        gsems = (g0, g1, g2, g3)
        ssems = (s0, s1, s2, s3)
        c = lax.axis_index("c")
        s = lax.axis_index("s")
        wid = s * NC + c
        pltpu.sync_copy(row_hbm.at[wid], row_v)
        pltpu.sync_copy(col_hbm.at[wid], col_v)
        for p in range(npass):
            pltpu.sync_copy(y_hbm.at[p, pl.ds(s * RPT, RPT)],
                            ystage.at[pl.ds(s * RPT, RPT)])
            pltpu.sync_copy(zeros_hbm.at[pl.ds(s * RPT, RPT)],
                            acc.at[pl.ds(s * RPT, RPT)])
            plsc.subcore_barrier()
            _prop_pass(ystage, col_v, out_hbm.at[c, p], bufs, acc,
                       gsems, ssems, row_v, s, c)
    return body


def _sc_propagate(y, row_tiles, col_tiles, zeros, width, npass):
    return pl.kernel(
        _make_prop_body(npass),
        out_type=jax.ShapeDtypeStruct((NC, npass, NPAD, width), jnp.float32),
        mesh=_mesh(),
        scratch_types=[
            pltpu.VMEM((NCHUNK, CHUNK), jnp.int32),
            pltpu.VMEM((NCHUNK, CHUNK), jnp.int32),
        ] + [pltpu.VMEM((CHUNK, width), jnp.float32)] * NBUF + [
            pltpu.VMEM_SHARED((NPAD, width), jnp.float32),
            pltpu.VMEM_SHARED((NPAD, width), jnp.float32),
        ] + [pltpu.SemaphoreType.DMA] * (2 * NBUF),
        compiler_params=_SC_PARAMS,
    )(y, row_tiles, col_tiles, zeros)


def _dinv(degp_ref):
    deg = degp_ref[0, :, 0:1] + degp_ref[1, :, 0:1] + 1.0
    return lax.rsqrt(deg)


def _lin1_body(x_ref, w_ref, degp_ref, y_ref):
    y = jnp.dot(x_ref[...], w_ref[...],
                preferred_element_type=jnp.float32) * _dinv(degp_ref)
    y_ref[0] = y[:, :HID // 2]
    y_ref[1] = y[:, HID // 2:]


def _tc_lin1(xpad, W1, degp):
    grid = (NPAD // RB,)
    return pl.pallas_call(
        _lin1_body,
        grid=grid,
        in_specs=[
            pl.BlockSpec((RB, F_IN), lambda i: (i, 0)),
            pl.BlockSpec((F_IN, HID), lambda i: (0, 0)),
            pl.BlockSpec((NC, RB, DEG_W), lambda i: (0, i, 0)),
        ],
        out_specs=pl.BlockSpec((2, RB, HID // 2), lambda i: (0, i, 0)),
        out_shape=jax.ShapeDtypeStruct((2, NPAD, HID // 2), jnp.float32),
    )(xpad, W1, degp)


def _lin2_body(sp_ref, y1_ref, degp_ref, b1_ref, w2_ref, y2_ref):
    dinv = _dinv(degp_ref)
    tot = sp_ref[0] + sp_ref[1] + y1_ref[...]
    pre = dinv * jnp.concatenate([tot[0], tot[1]], axis=1) + b1_ref[...]
    h = jnp.maximum(pre, 0.0)
    y2_ref[...] = jnp.dot(h, w2_ref[...],
                          preferred_element_type=jnp.float32) * dinv


def _tc_lin2(s1, y1, degp, b1r, W2p):
    grid = (NPAD // RB,)
    return pl.pallas_call(
        _lin2_body,
        grid=grid,
        in_specs=[
            pl.BlockSpec((NC, 2, RB, HID // 2), lambda i: (0, 0, i, 0)),
            pl.BlockSpec((2, RB, HID // 2), lambda i: (0, i, 0)),
            pl.BlockSpec((NC, RB, DEG_W), lambda i: (0, i, 0)),
            pl.BlockSpec((1, HID), lambda i: (0, 0)),
            pl.BlockSpec((HID, CPAD), lambda i: (0, 0)),
        ],
        out_specs=pl.BlockSpec((RB, CPAD), lambda i: (i, 0)),
        out_shape=jax.ShapeDtypeStruct((NPAD, CPAD), jnp.float32),
    )(s1, y1, degp, b1r, W2p)


def _out_body(tp_ref, y2_ref, degp_ref, b2_ref, o_ref):
    dinv = _dinv(degp_ref)
    z = dinv * (tp_ref[0] + tp_ref[1] + y2_ref[...]) + b2_ref[...]
    colid = lax.broadcasted_iota(jnp.int32, z.shape, 1)
    z = jnp.where(colid < CLS, z, -1e30)
    m = jnp.max(z, axis=1, keepdims=True)
    lse = jnp.log(jnp.sum(jnp.exp(z - m), axis=1, keepdims=True)) + m
    o_ref[...] = z - lse


def _tc_out(t1, y2, degp, b2p):
    grid = (NPAD // RB,)
    return pl.pallas_call(
        _out_body,
        grid=grid,
        in_specs=[
            pl.BlockSpec((NC, RB, CPAD), lambda i: (0, i, 0)),
            pl.BlockSpec((RB, CPAD), lambda i: (i, 0)),
            pl.BlockSpec((NC, RB, DEG_W), lambda i: (0, i, 0)),
            pl.BlockSpec((1, CPAD), lambda i: (0, 0)),
        ],
        out_specs=pl.BlockSpec((RB, CPAD), lambda i: (i, 0)),
        out_shape=jax.ShapeDtypeStruct((NPAD, CPAD), jnp.float32),
    )(t1, y2, degp, b2p)


def kernel(x, edge_index, W1, b1, W2, b2):
    ei = edge_index.astype(jnp.int32)
    padn = NW * EPT - E
    rowp = jnp.concatenate(
        [ei[0], jnp.full((padn,), N, jnp.int32)]).reshape(NW, NCHUNK, CHUNK)
    colp = jnp.concatenate(
        [ei[1], jnp.full((padn,), N, jnp.int32)]).reshape(NW, NCHUNK, CHUNK)
    xpad = jnp.zeros((NPAD, F_IN), jnp.float32).at[:N].set(x)
    ones16 = jnp.ones((CHUNK, DEG_W), jnp.float32)
    z16 = jnp.zeros((NPAD, DEG_W), jnp.float32)
    z64 = jnp.zeros((NPAD, HID // 2), jnp.float32)
    z48 = jnp.zeros((NPAD, CPAD), jnp.float32)
    W2p = jnp.zeros((HID, CPAD), jnp.float32).at[:, :CLS].set(W2)
    b2p = jnp.zeros((1, CPAD), jnp.float32).at[0, :CLS].set(b2)
    b1r = b1.reshape(1, HID)

    degp = _sc_degree(colp, ones16, z16)
    y1 = _tc_lin1(xpad, W1, degp)
    s1 = _sc_propagate(y1, rowp, colp, z64, HID // 2, 2)
    y2 = _tc_lin2(s1, y1, degp, b1r, W2p)
    t1 = _sc_propagate(y2.reshape(1, NPAD, CPAD), rowp, colp, z48, CPAD, 1)
    out = _tc_out(t1.reshape(NC, NPAD, CPAD), y2, degp, b2p)
    return out[:N, :CLS]


# speedup vs baseline: 29.0332x; 1.3444x over previous
import jax
import jax.numpy as jnp
from jax import lax
from jax.experimental import pallas as pl
from jax.experimental.pallas import tpu as pltpu
from jax.experimental.pallas import tpu_sc as plsc

N = 10000
E = 320000
F_IN = 128
HID = 128
CLS = 40
CPAD = 48

NC, NS = 2, 16
NW = NC * NS
NPAD = 10112
RPT = NPAD // NS
CHUNK = 80
EPT = 10240
NCHUNK = EPT // CHUNK
DEG_W = 16
RB = 1264


def _mesh():
    return plsc.VectorSubcoreMesh(
        core_axis_name="c", subcore_axis_name="s",
        num_cores=NC, num_subcores=NS)


_SC_PARAMS = pltpu.CompilerParams(use_tc_tiling_on_sc=False)


def _deg_body(col_hbm, ones_hbm, zeros_hbm, out_hbm, col_v, ones_v, acc):
    c = lax.axis_index("c")
    s = lax.axis_index("s")
    wid = s * NC + c
    pltpu.sync_copy(zeros_hbm.at[pl.ds(s * RPT, RPT)],
                    acc.at[pl.ds(s * RPT, RPT)])
    pltpu.sync_copy(col_hbm.at[wid], col_v)
    pltpu.sync_copy(ones_hbm, ones_v)
    plsc.subcore_barrier()

    def body(j, carry):
        pltpu.sync_copy(ones_v, acc.at[col_v.at[j]], add=True)
        return carry

    lax.fori_loop(0, NCHUNK, body, 0)
    plsc.subcore_barrier()
    pltpu.sync_copy(acc.at[pl.ds(s * RPT, RPT)],
                    out_hbm.at[c, pl.ds(s * RPT, RPT)])


def _sc_degree(col_tiles, ones, zeros16):
    return pl.kernel(
        _deg_body,
        out_type=jax.ShapeDtypeStruct((NC, NPAD, DEG_W), jnp.float32),
        mesh=_mesh(),
        scratch_types=[
            pltpu.VMEM((NCHUNK, CHUNK), jnp.int32),
            pltpu.VMEM((CHUNK, DEG_W), jnp.float32),
            pltpu.VMEM_SHARED((NPAD, DEG_W), jnp.float32),
        ],
        compiler_params=_SC_PARAMS,
    )(col_tiles, ones, zeros16)


NBUF = 4


def _prop_pass(y2d, col_v, out2d, bufs, acc, gsems, ssems, row_v, s, c):
    for b in range(NBUF - 1):
        pltpu.async_copy(y2d.at[row_v.at[b]], bufs[b], gsems[b])

    def body(j2, carry):
        for b in range(NBUF):
            j = j2 * NBUF + b
            nb = (b + NBUF - 1) % NBUF
            pltpu.make_async_copy(y2d.at[row_v.at[j]], bufs[b],
                                  gsems[b]).wait()
            pltpu.async_copy(bufs[b], acc.at[col_v.at[j]], ssems[b], add=True)
            def _wait_prev(j=j, nb=nb):
                pltpu.make_async_copy(bufs[nb], acc.at[col_v.at[j]],
                                      ssems[nb]).wait()
            if b == 0:
                pl.when(j2 > 0)(_wait_prev)
            else:
                _wait_prev()
            jn = jnp.minimum(j + NBUF - 1, NCHUNK - 1)
            pltpu.async_copy(y2d.at[row_v.at[jn]], bufs[nb], gsems[nb])
        return carry

    lax.fori_loop(0, NCHUNK // NBUF, body, 0)
    lb = (NCHUNK - 1) % NBUF
    pltpu.make_async_copy(bufs[lb], acc.at[col_v.at[0]], ssems[lb]).wait()
    for b in range(NBUF - 1):
        pltpu.make_async_copy(y2d.at[row_v.at[0]], bufs[b], gsems[b]).wait()
    plsc.subcore_barrier()
    pltpu.sync_copy(acc.at[pl.ds(s * RPT, RPT)],
                    out2d.at[pl.ds(s * RPT, RPT)])


def _make_prop_body(npass):
    def body(y_hbm, row_hbm, col_hbm, zeros_hbm, out_hbm,
             row_v, col_v, b0, b1, b2, b3, ystage, acc,
             g0, g1, g2, g3, s0, s1, s2, s3):
        bufs = (b0, b1, b2, b3)
        gsems = (g0, g1, g2, g3)
        ssems = (s0, s1, s2, s3)
        c = lax.axis_index("c")
        s = lax.axis_index("s")
        wid = s * NC + c
        pltpu.sync_copy(row_hbm.at[wid], row_v)
        pltpu.sync_copy(col_hbm.at[wid], col_v)
        for p in range(npass):
            pltpu.sync_copy(y_hbm.at[p, pl.ds(s * RPT, RPT)],
                            ystage.at[pl.ds(s * RPT, RPT)])
            pltpu.sync_copy(zeros_hbm.at[pl.ds(s * RPT, RPT)],
                            acc.at[pl.ds(s * RPT, RPT)])
            plsc.subcore_barrier()
            _prop_pass(ystage, col_v, out_hbm.at[c, p], bufs, acc,
                       gsems, ssems, row_v, s, c)
    return body


def _sc_propagate(y, row_tiles, col_tiles, zeros, width, npass):
    return pl.kernel(
        _make_prop_body(npass),
        out_type=jax.ShapeDtypeStruct((NC, npass, NPAD, width), jnp.float32),
        mesh=_mesh(),
        scratch_types=[
            pltpu.VMEM((NCHUNK, CHUNK), jnp.int32),
            pltpu.VMEM((NCHUNK, CHUNK), jnp.int32),
        ] + [pltpu.VMEM((CHUNK, width), jnp.float32)] * NBUF + [
            pltpu.VMEM_SHARED((NPAD, width), jnp.float32),
            pltpu.VMEM_SHARED((NPAD, width), jnp.float32),
        ] + [pltpu.SemaphoreType.DMA] * (2 * NBUF),
        compiler_params=_SC_PARAMS,
    )(y, row_tiles, col_tiles, zeros)


def _dinv(degp_ref):
    deg = degp_ref[0, :, 0:1] + degp_ref[1, :, 0:1] + 1.0
    return lax.rsqrt(deg)


def _mm1_body(x_ref, w_ref, xw_ref):
    xw_ref[...] = jnp.dot(x_ref[...], w_ref[...],
                          preferred_element_type=jnp.float32)


def _tc_mm1(xpad, W1):
    grid = (NPAD // RB,)
    return pl.pallas_call(
        _mm1_body,
        grid=grid,
        in_specs=[
            pl.BlockSpec((RB, F_IN), lambda i: (i, 0)),
            pl.BlockSpec((F_IN, HID), lambda i: (0, 0)),
        ],
        out_specs=pl.BlockSpec((RB, HID), lambda i: (i, 0)),
        out_shape=jax.ShapeDtypeStruct((NPAD, HID), jnp.float32),
    )(xpad, W1)


def _scale1_body(xw_ref, degp_ref, y_ref):
    y = xw_ref[...] * _dinv(degp_ref)
    y_ref[0] = y[:, :HID // 2]
    y_ref[1] = y[:, HID // 2:]


def _tc_scale1(xw, degp):
    grid = (NPAD // RB,)
    return pl.pallas_call(
        _scale1_body,
        grid=grid,
        in_specs=[
            pl.BlockSpec((RB, HID), lambda i: (i, 0)),
            pl.BlockSpec((NC, RB, DEG_W), lambda i: (0, i, 0)),
        ],
        out_specs=pl.BlockSpec((2, RB, HID // 2), lambda i: (0, i, 0)),
        out_shape=jax.ShapeDtypeStruct((2, NPAD, HID // 2), jnp.float32),
    )(xw, degp)


def _lin2_body(sp_ref, y1_ref, degp_ref, b1_ref, w2_ref, y2_ref):
    dinv = _dinv(degp_ref)
    tot = sp_ref[0] + sp_ref[1] + y1_ref[...]
    pre = dinv * jnp.concatenate([tot[0], tot[1]], axis=1) + b1_ref[...]
    h = jnp.maximum(pre, 0.0)
    y2_ref[...] = jnp.dot(h, w2_ref[...],
                          preferred_element_type=jnp.float32) * dinv


def _tc_lin2(s1, y1, degp, b1r, W2p):
    grid = (NPAD // RB,)
    return pl.pallas_call(
        _lin2_body,
        grid=grid,
        in_specs=[
            pl.BlockSpec((NC, 2, RB, HID // 2), lambda i: (0, 0, i, 0)),
            pl.BlockSpec((2, RB, HID // 2), lambda i: (0, i, 0)),
            pl.BlockSpec((NC, RB, DEG_W), lambda i: (0, i, 0)),
            pl.BlockSpec((1, HID), lambda i: (0, 0)),
            pl.BlockSpec((HID, CPAD), lambda i: (0, 0)),
        ],
        out_specs=pl.BlockSpec((RB, CPAD), lambda i: (i, 0)),
        out_shape=jax.ShapeDtypeStruct((NPAD, CPAD), jnp.float32),
    )(s1, y1, degp, b1r, W2p)


def _out_body(tp_ref, y2_ref, degp_ref, b2_ref, o_ref):
    dinv = _dinv(degp_ref)
    z = dinv * (tp_ref[0] + tp_ref[1] + y2_ref[...]) + b2_ref[...]
    colid = lax.broadcasted_iota(jnp.int32, z.shape, 1)
    z = jnp.where(colid < CLS, z, -1e30)
    m = jnp.max(z, axis=1, keepdims=True)
    lse = jnp.log(jnp.sum(jnp.exp(z - m), axis=1, keepdims=True)) + m
    o_ref[...] = z - lse


def _tc_out(t1, y2, degp, b2p):
    grid = (NPAD // RB,)
    return pl.pallas_call(
        _out_body,
        grid=grid,
        in_specs=[
            pl.BlockSpec((NC, RB, CPAD), lambda i: (0, i, 0)),
            pl.BlockSpec((RB, CPAD), lambda i: (i, 0)),
            pl.BlockSpec((NC, RB, DEG_W), lambda i: (0, i, 0)),
            pl.BlockSpec((1, CPAD), lambda i: (0, 0)),
        ],
        out_specs=pl.BlockSpec((RB, CPAD), lambda i: (i, 0)),
        out_shape=jax.ShapeDtypeStruct((NPAD, CPAD), jnp.float32),
    )(t1, y2, degp, b2p)


def kernel(x, edge_index, W1, b1, W2, b2):
    ei = edge_index.astype(jnp.int32)
    padn = NW * EPT - E
    rowp = jnp.concatenate(
        [ei[0], jnp.full((padn,), N, jnp.int32)]).reshape(NW, NCHUNK, CHUNK)
    colp = jnp.concatenate(
        [ei[1], jnp.full((padn,), N, jnp.int32)]).reshape(NW, NCHUNK, CHUNK)
    xpad = jnp.zeros((NPAD, F_IN), jnp.float32).at[:N].set(x)
    ones16 = jnp.ones((CHUNK, DEG_W), jnp.float32)
    z16 = jnp.zeros((NPAD, DEG_W), jnp.float32)
    z64 = jnp.zeros((NPAD, HID // 2), jnp.float32)
    z48 = jnp.zeros((NPAD, CPAD), jnp.float32)
    W2p = jnp.zeros((HID, CPAD), jnp.float32).at[:, :CLS].set(W2)
    b2p = jnp.zeros((1, CPAD), jnp.float32).at[0, :CLS].set(b2)
    b1r = b1.reshape(1, HID)

    degp = _sc_degree(colp, ones16, z16)
    xw = _tc_mm1(xpad, W1)
    y1 = _tc_scale1(xw, degp)
    s1 = _sc_propagate(y1, rowp, colp, z64, HID // 2, 2)
    y2 = _tc_lin2(s1, y1, degp, b1r, W2p)
    t1 = _sc_propagate(y2.reshape(1, NPAD, CPAD), rowp, colp, z48, CPAD, 1)
    out = _tc_out(t1.reshape(NC, NPAD, CPAD), y2, degp, b2p)
    return out[:N, :CLS]


# speedup vs baseline: 29.3235x; 1.0100x over previous
import jax
import jax.numpy as jnp
from jax import lax
from jax.experimental import pallas as pl
from jax.experimental.pallas import tpu as pltpu
from jax.experimental.pallas import tpu_sc as plsc

N = 10000
E = 320000
F_IN = 128
HID = 128
CLS = 40
CPAD = 48

NC, NS = 2, 16
NW = NC * NS
NPAD = 10112
RPT = NPAD // NS
CHUNK = 80
EPT = 10240
NCHUNK = EPT // CHUNK
DEG_W = 16
RB = 1264


def _mesh():
    return plsc.VectorSubcoreMesh(
        core_axis_name="c", subcore_axis_name="s",
        num_cores=NC, num_subcores=NS)


_SC_PARAMS = pltpu.CompilerParams(use_tc_tiling_on_sc=False)


def _deg_body(col_hbm, ones_hbm, zeros_hbm, out_hbm, col_v, ones_v, acc):
    c = lax.axis_index("c")
    s = lax.axis_index("s")
    wid = s * NC + c
    pltpu.sync_copy(zeros_hbm.at[pl.ds(s * RPT, RPT)],
                    acc.at[pl.ds(s * RPT, RPT)])
    pltpu.sync_copy(col_hbm.at[wid], col_v)
    pltpu.sync_copy(ones_hbm, ones_v)
    plsc.subcore_barrier()

    def body(j, carry):
        pltpu.sync_copy(ones_v, acc.at[col_v.at[j]], add=True)
        return carry

    lax.fori_loop(0, NCHUNK, body, 0)
    plsc.subcore_barrier()
    pltpu.sync_copy(acc.at[pl.ds(s * RPT, RPT)],
                    out_hbm.at[c, pl.ds(s * RPT, RPT)])


def _sc_degree(col_tiles, ones, zeros16):
    return pl.kernel(
        _deg_body,
        out_type=jax.ShapeDtypeStruct((NC, NPAD, DEG_W), jnp.float32),
        mesh=_mesh(),
        scratch_types=[
            pltpu.VMEM((NCHUNK, CHUNK), jnp.int32),
            pltpu.VMEM((CHUNK, DEG_W), jnp.float32),
            pltpu.VMEM_SHARED((NPAD, DEG_W), jnp.float32),
        ],
        compiler_params=_SC_PARAMS,
    )(col_tiles, ones, zeros16)


NBUF = 4


def _prop_pass(y2d, col_v, out2d, bufs, acc, gsems, ssems, row_v, s, c):
    for b in range(NBUF - 1):
        pltpu.async_copy(y2d.at[row_v.at[b]], bufs[b], gsems[b])

    def body(j2, carry):
        for b in range(NBUF):
            j = j2 * NBUF + b
            nb = (b + NBUF - 1) % NBUF
            pltpu.make_async_copy(y2d.at[row_v.at[j]], bufs[b],
                                  gsems[b]).wait()
            pltpu.async_copy(bufs[b], acc.at[col_v.at[j]], ssems[b], add=True)
            def _wait_prev(j=j, nb=nb):
                pltpu.make_async_copy(bufs[nb], acc.at[col_v.at[j]],
                                      ssems[nb]).wait()
            if b == 0:
                pl.when(j2 > 0)(_wait_prev)
            else:
                _wait_prev()
            jn = jnp.minimum(j + NBUF - 1, NCHUNK - 1)
            pltpu.async_copy(y2d.at[row_v.at[jn]], bufs[nb], gsems[nb])
        return carry

    lax.fori_loop(0, NCHUNK // NBUF, body, 0)
    lb = (NCHUNK - 1) % NBUF
    pltpu.make_async_copy(bufs[lb], acc.at[col_v.at[0]], ssems[lb]).wait()
    for b in range(NBUF - 1):
        pltpu.make_async_copy(y2d.at[row_v.at[0]], bufs[b], gsems[b]).wait()
    plsc.subcore_barrier()
    pltpu.sync_copy(acc.at[pl.ds(s * RPT, RPT)],
                    out2d.at[pl.ds(s * RPT, RPT)])


def _make_prop_body(npass):
    def body(y_hbm, row_hbm, col_hbm, zeros_hbm, out_hbm,
             row_v, col_v, b0, b1, b2, b3, ystage, acc,
             g0, g1, g2, g3, s0, s1, s2, s3):
        bufs = (b0, b1, b2, b3)
        gsems = (g0, g1, g2, g3)
        ssems = (s0, s1, s2, s3)
        c = lax.axis_index("c")
        s = lax.axis_index("s")
        wid = s * NC + c
        pltpu.sync_copy(row_hbm.at[wid], row_v)
        pltpu.sync_copy(col_hbm.at[wid], col_v)
        for p in range(npass):
            pltpu.sync_copy(y_hbm.at[p, pl.ds(s * RPT, RPT)],
                            ystage.at[pl.ds(s * RPT, RPT)])

            @pl.when(c == 0)
            def _():
                pltpu.sync_copy(y_hbm.at[p, pl.ds(s * RPT, RPT)],
                                acc.at[pl.ds(s * RPT, RPT)])

            @pl.when(c != 0)
            def _():
                pltpu.sync_copy(zeros_hbm.at[pl.ds(s * RPT, RPT)],
                                acc.at[pl.ds(s * RPT, RPT)])

            plsc.subcore_barrier()
            _prop_pass(ystage, col_v, out_hbm.at[c, p], bufs, acc,
                       gsems, ssems, row_v, s, c)
    return body


def _sc_propagate(y, row_tiles, col_tiles, zeros, width, npass):
    return pl.kernel(
        _make_prop_body(npass),
        out_type=jax.ShapeDtypeStruct((NC, npass, NPAD, width), jnp.float32),
        mesh=_mesh(),
        scratch_types=[
            pltpu.VMEM((NCHUNK, CHUNK), jnp.int32),
            pltpu.VMEM((NCHUNK, CHUNK), jnp.int32),
        ] + [pltpu.VMEM((CHUNK, width), jnp.float32)] * NBUF + [
            pltpu.VMEM_SHARED((NPAD, width), jnp.float32),
            pltpu.VMEM_SHARED((NPAD, width), jnp.float32),
        ] + [pltpu.SemaphoreType.DMA] * (2 * NBUF),
        compiler_params=_SC_PARAMS,
    )(y, row_tiles, col_tiles, zeros)


def _dinv(degp_ref):
    deg = degp_ref[0, :, 0:1] + degp_ref[1, :, 0:1] + 1.0
    return lax.rsqrt(deg)


def _mm1_body(x_ref, w_ref, xw_ref):
    xw_ref[...] = jnp.dot(x_ref[...], w_ref[...],
                          preferred_element_type=jnp.float32)


def _tc_mm1(xpad, W1):
    grid = (NPAD // RB,)
    return pl.pallas_call(
        _mm1_body,
        grid=grid,
        in_specs=[
            pl.BlockSpec((RB, F_IN), lambda i: (i, 0)),
            pl.BlockSpec((F_IN, HID), lambda i: (0, 0)),
        ],
        out_specs=pl.BlockSpec((RB, HID), lambda i: (i, 0)),
        out_shape=jax.ShapeDtypeStruct((NPAD, HID), jnp.float32),
    )(xpad, W1)


def _scale1_body(xw_ref, degp_ref, y_ref):
    y = xw_ref[...] * _dinv(degp_ref)
    y_ref[0] = y[:, :HID // 2]
    y_ref[1] = y[:, HID // 2:]


def _tc_scale1(xw, degp):
    grid = (NPAD // RB,)
    return pl.pallas_call(
        _scale1_body,
        grid=grid,
        in_specs=[
            pl.BlockSpec((RB, HID), lambda i: (i, 0)),
            pl.BlockSpec((NC, RB, DEG_W), lambda i: (0, i, 0)),
        ],
        out_specs=pl.BlockSpec((2, RB, HID // 2), lambda i: (0, i, 0)),
        out_shape=jax.ShapeDtypeStruct((2, NPAD, HID // 2), jnp.float32),
    )(xw, degp)


def _lin2_body(sp_ref, degp_ref, b1_ref, w2_ref, y2_ref):
    dinv = _dinv(degp_ref)
    tot = sp_ref[0] + sp_ref[1]
    pre = dinv * jnp.concatenate([tot[0], tot[1]], axis=1) + b1_ref[...]
    h = jnp.maximum(pre, 0.0)
    y2_ref[...] = jnp.dot(h, w2_ref[...],
                          preferred_element_type=jnp.float32) * dinv


def _tc_lin2(s1, degp, b1r, W2p):
    grid = (NPAD // RB,)
    return pl.pallas_call(
        _lin2_body,
        grid=grid,
        in_specs=[
            pl.BlockSpec((NC, 2, RB, HID // 2), lambda i: (0, 0, i, 0)),
            pl.BlockSpec((NC, RB, DEG_W), lambda i: (0, i, 0)),
            pl.BlockSpec((1, HID), lambda i: (0, 0)),
            pl.BlockSpec((HID, CPAD), lambda i: (0, 0)),
        ],
        out_specs=pl.BlockSpec((RB, CPAD), lambda i: (i, 0)),
        out_shape=jax.ShapeDtypeStruct((NPAD, CPAD), jnp.float32),
    )(s1, degp, b1r, W2p)


def _out_body(tp_ref, degp_ref, b2_ref, o_ref):
    dinv = _dinv(degp_ref)
    z = dinv * (tp_ref[0] + tp_ref[1]) + b2_ref[...]
    colid = lax.broadcasted_iota(jnp.int32, z.shape, 1)
    z = jnp.where(colid < CLS, z, -1e30)
    m = jnp.max(z, axis=1, keepdims=True)
    lse = jnp.log(jnp.sum(jnp.exp(z - m), axis=1, keepdims=True)) + m
    o_ref[...] = z - lse


def _tc_out(t1, degp, b2p):
    grid = (NPAD // RB,)
    return pl.pallas_call(
        _out_body,
        grid=grid,
        in_specs=[
            pl.BlockSpec((NC, RB, CPAD), lambda i: (0, i, 0)),
            pl.BlockSpec((NC, RB, DEG_W), lambda i: (0, i, 0)),
            pl.BlockSpec((1, CPAD), lambda i: (0, 0)),
        ],
        out_specs=pl.BlockSpec((RB, CPAD), lambda i: (i, 0)),
        out_shape=jax.ShapeDtypeStruct((NPAD, CPAD), jnp.float32),
    )(t1, degp, b2p)


def kernel(x, edge_index, W1, b1, W2, b2):
    ei = edge_index.astype(jnp.int32)
    padn = NW * EPT - E
    rowp = jnp.concatenate(
        [ei[0], jnp.full((padn,), N, jnp.int32)]).reshape(NW, NCHUNK, CHUNK)
    colp = jnp.concatenate(
        [ei[1], jnp.full((padn,), N, jnp.int32)]).reshape(NW, NCHUNK, CHUNK)
    xpad = jnp.zeros((NPAD, F_IN), jnp.float32).at[:N].set(x)
    ones16 = jnp.ones((CHUNK, DEG_W), jnp.float32)
    z16 = jnp.zeros((NPAD, DEG_W), jnp.float32)
    z64 = jnp.zeros((NPAD, HID // 2), jnp.float32)
    z48 = jnp.zeros((NPAD, CPAD), jnp.float32)
    W2p = jnp.zeros((HID, CPAD), jnp.float32).at[:, :CLS].set(W2)
    b2p = jnp.zeros((1, CPAD), jnp.float32).at[0, :CLS].set(b2)
    b1r = b1.reshape(1, HID)

    degp = _sc_degree(colp, ones16, z16)
    xw = _tc_mm1(xpad, W1)
    y1 = _tc_scale1(xw, degp)
    s1 = _sc_propagate(y1, rowp, colp, z64, HID // 2, 2)
    y2 = _tc_lin2(s1, degp, b1r, W2p)
    t1 = _sc_propagate(y2.reshape(1, NPAD, CPAD), rowp, colp, z48, CPAD, 1)
    out = _tc_out(t1.reshape(NC, NPAD, CPAD), degp, b2p)
    return out[:N, :CLS]


# speedup vs baseline: 29.3669x; 1.0015x over previous
import jax
import jax.numpy as jnp
from jax import lax
from jax.experimental import pallas as pl
from jax.experimental.pallas import tpu as pltpu
from jax.experimental.pallas import tpu_sc as plsc

N = 10000
E = 320000
F_IN = 128
HID = 128
CLS = 40
CPAD = 48

NC, NS = 2, 16
NW = NC * NS
NPAD = 10112
RPT = NPAD // NS
CHUNK = 80
EPT = 10240
NCHUNK = EPT // CHUNK
DEG_W = 16
RB = 1264


def _mesh():
    return plsc.VectorSubcoreMesh(
        core_axis_name="c", subcore_axis_name="s",
        num_cores=NC, num_subcores=NS)


_SC_PARAMS = pltpu.CompilerParams(use_tc_tiling_on_sc=False)


DEG_K = 8


def _deg_body(col_hbm, ones_hbm, zeros_hbm, out_hbm, col_v, ones_v, acc,
              dsem):
    c = lax.axis_index("c")
    s = lax.axis_index("s")
    wid = s * NC + c
    pltpu.sync_copy(zeros_hbm.at[pl.ds(s * RPT, RPT)],
                    acc.at[pl.ds(s * RPT, RPT)])
    pltpu.sync_copy(col_hbm.at[wid], col_v)
    pltpu.sync_copy(ones_hbm, ones_v)
    plsc.subcore_barrier()

    def body(j2, carry):
        for b in range(DEG_K):
            pltpu.async_copy(ones_v, acc.at[col_v.at[j2 * DEG_K + b]],
                             dsem, add=True)
        for b in range(DEG_K):
            pltpu.make_async_copy(ones_v, acc.at[col_v.at[0]], dsem).wait()
        return carry

    lax.fori_loop(0, NCHUNK // DEG_K, body, 0)
    plsc.subcore_barrier()
    pltpu.sync_copy(acc.at[pl.ds(s * RPT, RPT)],
                    out_hbm.at[c, pl.ds(s * RPT, RPT)])


def _sc_degree(col_tiles, ones, zeros16):
    return pl.kernel(
        _deg_body,
        out_type=jax.ShapeDtypeStruct((NC, NPAD, DEG_W), jnp.float32),
        mesh=_mesh(),
        scratch_types=[
            pltpu.VMEM((NCHUNK, CHUNK), jnp.int32),
            pltpu.VMEM((CHUNK, DEG_W), jnp.float32),
            pltpu.VMEM_SHARED((NPAD, DEG_W), jnp.float32),
            pltpu.SemaphoreType.DMA,
        ],
        compiler_params=_SC_PARAMS,
    )(col_tiles, ones, zeros16)


NBUF = 4


def _prop_pass(y2d, col_v, out2d, bufs, acc, gsems, ssems, row_v, s, c):
    for b in range(NBUF - 1):
        pltpu.async_copy(y2d.at[row_v.at[b]], bufs[b], gsems[b])

    def body(j2, carry):
        for b in range(NBUF):
            j = j2 * NBUF + b
            nb = (b + NBUF - 1) % NBUF
            pltpu.make_async_copy(y2d.at[row_v.at[j]], bufs[b],
                                  gsems[b]).wait()
            pltpu.async_copy(bufs[b], acc.at[col_v.at[j]], ssems[b], add=True)
            def _wait_prev(j=j, nb=nb):
                pltpu.make_async_copy(bufs[nb], acc.at[col_v.at[j]],
                                      ssems[nb]).wait()
            if b == 0:
                pl.when(j2 > 0)(_wait_prev)
            else:
                _wait_prev()
            jn = jnp.minimum(j + NBUF - 1, NCHUNK - 1)
            pltpu.async_copy(y2d.at[row_v.at[jn]], bufs[nb], gsems[nb])
        return carry

    lax.fori_loop(0, NCHUNK // NBUF, body, 0)
    lb = (NCHUNK - 1) % NBUF
    pltpu.make_async_copy(bufs[lb], acc.at[col_v.at[0]], ssems[lb]).wait()
    for b in range(NBUF - 1):
        pltpu.make_async_copy(y2d.at[row_v.at[0]], bufs[b], gsems[b]).wait()
    plsc.subcore_barrier()
    pltpu.sync_copy(acc.at[pl.ds(s * RPT, RPT)],
                    out2d.at[pl.ds(s * RPT, RPT)])


def _make_prop_body(npass):
    def body(y_hbm, row_hbm, col_hbm, zeros_hbm, out_hbm,
             row_v, col_v, b0, b1, b2, b3, ystage, acc,
             g0, g1, g2, g3, s0, s1, s2, s3):
        bufs = (b0, b1, b2, b3)
        gsems = (g0, g1, g2, g3)
        ssems = (s0, s1, s2, s3)
        c = lax.axis_index("c")
        s = lax.axis_index("s")
        wid = s * NC + c
        pltpu.sync_copy(row_hbm.at[wid], row_v)
        pltpu.sync_copy(col_hbm.at[wid], col_v)
        for p in range(npass):
            pltpu.sync_copy(y_hbm.at[p, pl.ds(s * RPT, RPT)],
                            ystage.at[pl.ds(s * RPT, RPT)])

            @pl.when(c == 0)
            def _():
                pltpu.sync_copy(y_hbm.at[p, pl.ds(s * RPT, RPT)],
                                acc.at[pl.ds(s * RPT, RPT)])

            @pl.when(c != 0)
            def _():
                pltpu.sync_copy(zeros_hbm.at[pl.ds(s * RPT, RPT)],
                                acc.at[pl.ds(s * RPT, RPT)])

            plsc.subcore_barrier()
            _prop_pass(ystage, col_v, out_hbm.at[c, p], bufs, acc,
                       gsems, ssems, row_v, s, c)
    return body


def _sc_propagate(y, row_tiles, col_tiles, zeros, width, npass):
    return pl.kernel(
        _make_prop_body(npass),
        out_type=jax.ShapeDtypeStruct((NC, npass, NPAD, width), jnp.float32),
        mesh=_mesh(),
        scratch_types=[
            pltpu.VMEM((NCHUNK, CHUNK), jnp.int32),
            pltpu.VMEM((NCHUNK, CHUNK), jnp.int32),
        ] + [pltpu.VMEM((CHUNK, width), jnp.float32)] * NBUF + [
            pltpu.VMEM_SHARED((NPAD, width), jnp.float32),
            pltpu.VMEM_SHARED((NPAD, width), jnp.float32),
        ] + [pltpu.SemaphoreType.DMA] * (2 * NBUF),
        compiler_params=_SC_PARAMS,
    )(y, row_tiles, col_tiles, zeros)


def _dinv(degp_ref):
    deg = degp_ref[0, :, 0:1] + degp_ref[1, :, 0:1] + 1.0
    return lax.rsqrt(deg)


def _mm1_body(x_ref, w_ref, xw_ref):
    xw_ref[...] = jnp.dot(x_ref[...], w_ref[...],
                          preferred_element_type=jnp.float32)


def _tc_mm1(xpad, W1):
    grid = (NPAD // RB,)
    return pl.pallas_call(
        _mm1_body,
        grid=grid,
        in_specs=[
            pl.BlockSpec((RB, F_IN), lambda i: (i, 0)),
            pl.BlockSpec((F_IN, HID), lambda i: (0, 0)),
        ],
        out_specs=pl.BlockSpec((RB, HID), lambda i: (i, 0)),
        out_shape=jax.ShapeDtypeStruct((NPAD, HID), jnp.float32),
    )(xpad, W1)


def _scale1_body(xw_ref, degp_ref, y_ref):
    y = xw_ref[...] * _dinv(degp_ref)
    y_ref[0] = y[:, :HID // 2]
    y_ref[1] = y[:, HID // 2:]


def _tc_scale1(xw, degp):
    grid = (NPAD // RB,)
    return pl.pallas_call(
        _scale1_body,
        grid=grid,
        in_specs=[
            pl.BlockSpec((RB, HID), lambda i: (i, 0)),
            pl.BlockSpec((NC, RB, DEG_W), lambda i: (0, i, 0)),
        ],
        out_specs=pl.BlockSpec((2, RB, HID // 2), lambda i: (0, i, 0)),
        out_shape=jax.ShapeDtypeStruct((2, NPAD, HID // 2), jnp.float32),
    )(xw, degp)


def _lin2_body(sp_ref, degp_ref, b1_ref, w2_ref, y2_ref):
    dinv = _dinv(degp_ref)
    tot = sp_ref[0] + sp_ref[1]
    pre = dinv * jnp.concatenate([tot[0], tot[1]], axis=1) + b1_ref[...]
    h = jnp.maximum(pre, 0.0)
    y2_ref[...] = jnp.dot(h, w2_ref[...],
                          preferred_element_type=jnp.float32) * dinv


def _tc_lin2(s1, degp, b1r, W2p):
    grid = (NPAD // RB,)
    return pl.pallas_call(
        _lin2_body,
        grid=grid,
        in_specs=[
            pl.BlockSpec((NC, 2, RB, HID // 2), lambda i: (0, 0, i, 0)),
            pl.BlockSpec((NC, RB, DEG_W), lambda i: (0, i, 0)),
            pl.BlockSpec((1, HID), lambda i: (0, 0)),
            pl.BlockSpec((HID, CPAD), lambda i: (0, 0)),
        ],
        out_specs=pl.BlockSpec((RB, CPAD), lambda i: (i, 0)),
        out_shape=jax.ShapeDtypeStruct((NPAD, CPAD), jnp.float32),
    )(s1, degp, b1r, W2p)


def _out_body(tp_ref, degp_ref, b2_ref, o_ref):
    dinv = _dinv(degp_ref)
    z = dinv * (tp_ref[0] + tp_ref[1]) + b2_ref[...]
    colid = lax.broadcasted_iota(jnp.int32, z.shape, 1)
    z = jnp.where(colid < CLS, z, -1e30)
    m = jnp.max(z, axis=1, keepdims=True)
    lse = jnp.log(jnp.sum(jnp.exp(z - m), axis=1, keepdims=True)) + m
    o_ref[...] = z - lse


def _tc_out(t1, degp, b2p):
    grid = (NPAD // RB,)
    return pl.pallas_call(
        _out_body,
        grid=grid,
        in_specs=[
            pl.BlockSpec((NC, RB, CPAD), lambda i: (0, i, 0)),
            pl.BlockSpec((NC, RB, DEG_W), lambda i: (0, i, 0)),
            pl.BlockSpec((1, CPAD), lambda i: (0, 0)),
        ],
        out_specs=pl.BlockSpec((RB, CPAD), lambda i: (i, 0)),
        out_shape=jax.ShapeDtypeStruct((NPAD, CPAD), jnp.float32),
    )(t1, degp, b2p)


def kernel(x, edge_index, W1, b1, W2, b2):
    ei = edge_index.astype(jnp.int32)
    padn = NW * EPT - E
    rowp = jnp.concatenate(
        [ei[0], jnp.full((padn,), N, jnp.int32)]).reshape(NW, NCHUNK, CHUNK)
    colp = jnp.concatenate(
        [ei[1], jnp.full((padn,), N, jnp.int32)]).reshape(NW, NCHUNK, CHUNK)
    xpad = jnp.zeros((NPAD, F_IN), jnp.float32).at[:N].set(x)
    ones16 = jnp.ones((CHUNK, DEG_W), jnp.float32)
    z16 = jnp.zeros((NPAD, DEG_W), jnp.float32)
    z64 = jnp.zeros((NPAD, HID // 2), jnp.float32)
    z48 = jnp.zeros((NPAD, CPAD), jnp.float32)
    W2p = jnp.zeros((HID, CPAD), jnp.float32).at[:, :CLS].set(W2)
    b2p = jnp.zeros((1, CPAD), jnp.float32).at[0, :CLS].set(b2)
    b1r = b1.reshape(1, HID)

    degp = _sc_degree(colp, ones16, z16)
    xw = _tc_mm1(xpad, W1)
    y1 = _tc_scale1(xw, degp)
    s1 = _sc_propagate(y1, rowp, colp, z64, HID // 2, 2)
    y2 = _tc_lin2(s1, degp, b1r, W2p)
    t1 = _sc_propagate(y2.reshape(1, NPAD, CPAD), rowp, colp, z48, CPAD, 1)
    out = _tc_out(t1.reshape(NC, NPAD, CPAD), degp, b2p)
    return out[:N, :CLS]


# speedup vs baseline: 29.3734x; 1.0002x over previous
import jax
import jax.numpy as jnp
from jax import lax
from jax.experimental import pallas as pl
from jax.experimental.pallas import tpu as pltpu
from jax.experimental.pallas import tpu_sc as plsc

N = 10000
E = 320000
F_IN = 128
HID = 128
CLS = 40
CPAD = 48

NC, NS = 2, 16
NW = NC * NS
NPAD = 10112
RPT = NPAD // NS
CHUNK = 80
EPT = 10240
NCHUNK = EPT // CHUNK
DEG_W = 16
RB = 1264


def _mesh():
    return plsc.VectorSubcoreMesh(
        core_axis_name="c", subcore_axis_name="s",
        num_cores=NC, num_subcores=NS)


_SC_PARAMS = pltpu.CompilerParams(use_tc_tiling_on_sc=False)


DEG_K = 8
DCHUNK = 128
DNCHUNK = EPT // DCHUNK


def _deg_body(col_hbm, ones_hbm, zeros_hbm, out_hbm, col_v, ones_v, acc,
              dsem):
    c = lax.axis_index("c")
    s = lax.axis_index("s")
    wid = s * NC + c
    pltpu.sync_copy(zeros_hbm.at[pl.ds(s * RPT, RPT)],
                    acc.at[pl.ds(s * RPT, RPT)])
    pltpu.sync_copy(col_hbm.at[wid], col_v)
    pltpu.sync_copy(ones_hbm, ones_v)
    plsc.subcore_barrier()

    def body(j2, carry):
        for b in range(DEG_K):
            pltpu.async_copy(ones_v, acc.at[col_v.at[j2 * DEG_K + b]],
                             dsem, add=True)
        for b in range(DEG_K):
            pltpu.make_async_copy(ones_v, acc.at[col_v.at[0]], dsem).wait()
        return carry

    lax.fori_loop(0, DNCHUNK // DEG_K, body, 0)
    plsc.subcore_barrier()
    pltpu.sync_copy(acc.at[pl.ds(s * RPT, RPT)],
                    out_hbm.at[c, pl.ds(s * RPT, RPT)])


def _sc_degree(col_tiles, ones, zeros16):
    return pl.kernel(
        _deg_body,
        out_type=jax.ShapeDtypeStruct((NC, NPAD, DEG_W), jnp.float32),
        mesh=_mesh(),
        scratch_types=[
            pltpu.VMEM((DNCHUNK, DCHUNK), jnp.int32),
            pltpu.VMEM((DCHUNK, DEG_W), jnp.float32),
            pltpu.VMEM_SHARED((NPAD, DEG_W), jnp.float32),
            pltpu.SemaphoreType.DMA,
        ],
        compiler_params=_SC_PARAMS,
    )(col_tiles, ones, zeros16)


NBUF = 4


def _prop_pass(y2d, col_v, out2d, bufs, acc, gsems, ssems, row_v, s, c):
    for b in range(NBUF - 1):
        pltpu.async_copy(y2d.at[row_v.at[b]], bufs[b], gsems[b])

    def body(j2, carry):
        for b in range(NBUF):
            j = j2 * NBUF + b
            nb = (b + NBUF - 1) % NBUF
            pltpu.make_async_copy(y2d.at[row_v.at[j]], bufs[b],
                                  gsems[b]).wait()
            pltpu.async_copy(bufs[b], acc.at[col_v.at[j]], ssems[b], add=True)
            def _wait_prev(j=j, nb=nb):
                pltpu.make_async_copy(bufs[nb], acc.at[col_v.at[j]],
                                      ssems[nb]).wait()
            if b == 0:
                pl.when(j2 > 0)(_wait_prev)
            else:
                _wait_prev()
            jn = jnp.minimum(j + NBUF - 1, NCHUNK - 1)
            pltpu.async_copy(y2d.at[row_v.at[jn]], bufs[nb], gsems[nb])
        return carry

    lax.fori_loop(0, NCHUNK // NBUF, body, 0)
    lb = (NCHUNK - 1) % NBUF
    pltpu.make_async_copy(bufs[lb], acc.at[col_v.at[0]], ssems[lb]).wait()
    for b in range(NBUF - 1):
        pltpu.make_async_copy(y2d.at[row_v.at[0]], bufs[b], gsems[b]).wait()
    plsc.subcore_barrier()
    pltpu.sync_copy(acc.at[pl.ds(s * RPT, RPT)],
                    out2d.at[pl.ds(s * RPT, RPT)])


def _make_prop_body(npass):
    def body(y_hbm, row_hbm, col_hbm, zeros_hbm, out_hbm,
             row_v, col_v, b0, b1, b2, b3, ystage, acc,
             g0, g1, g2, g3, s0, s1, s2, s3):
        bufs = (b0, b1, b2, b3)
        gsems = (g0, g1, g2, g3)
        ssems = (s0, s1, s2, s3)
        c = lax.axis_index("c")
        s = lax.axis_index("s")
        wid = s * NC + c
        pltpu.sync_copy(row_hbm.at[wid], row_v)
        pltpu.sync_copy(col_hbm.at[wid], col_v)
        for p in range(npass):
            pltpu.sync_copy(y_hbm.at[p, pl.ds(s * RPT, RPT)],
                            ystage.at[pl.ds(s * RPT, RPT)])

            @pl.when(c == 0)
            def _():
                pltpu.sync_copy(y_hbm.at[p, pl.ds(s * RPT, RPT)],
                                acc.at[pl.ds(s * RPT, RPT)])

            @pl.when(c != 0)
            def _():
                pltpu.sync_copy(zeros_hbm.at[pl.ds(s * RPT, RPT)],
                                acc.at[pl.ds(s * RPT, RPT)])

            plsc.subcore_barrier()
            _prop_pass(ystage, col_v, out_hbm.at[c, p], bufs, acc,
                       gsems, ssems, row_v, s, c)
    return body


def _sc_propagate(y, row_tiles, col_tiles, zeros, width, npass):
    return pl.kernel(
        _make_prop_body(npass),
        out_type=jax.ShapeDtypeStruct((NC, npass, NPAD, width), jnp.float32),
        mesh=_mesh(),
        scratch_types=[
            pltpu.VMEM((NCHUNK, CHUNK), jnp.int32),
            pltpu.VMEM((NCHUNK, CHUNK), jnp.int32),
        ] + [pltpu.VMEM((CHUNK, width), jnp.float32)] * NBUF + [
            pltpu.VMEM_SHARED((NPAD, width), jnp.float32),
            pltpu.VMEM_SHARED((NPAD, width), jnp.float32),
        ] + [pltpu.SemaphoreType.DMA] * (2 * NBUF),
        compiler_params=_SC_PARAMS,
    )(y, row_tiles, col_tiles, zeros)


def _dinv(degp_ref):
    deg = degp_ref[0, :, 0:1] + degp_ref[1, :, 0:1] + 1.0
    return lax.rsqrt(deg)


def _mm1_body(x_ref, w_ref, xw_ref):
    xw_ref[...] = jnp.dot(x_ref[...], w_ref[...],
                          preferred_element_type=jnp.float32)


def _tc_mm1(xpad, W1):
    grid = (NPAD // RB,)
    return pl.pallas_call(
        _mm1_body,
        grid=grid,
        in_specs=[
            pl.BlockSpec((RB, F_IN), lambda i: (i, 0)),
            pl.BlockSpec((F_IN, HID), lambda i: (0, 0)),
        ],
        out_specs=pl.BlockSpec((RB, HID), lambda i: (i, 0)),
        out_shape=jax.ShapeDtypeStruct((NPAD, HID), jnp.float32),
    )(xpad, W1)


def _scale1_body(xw_ref, degp_ref, y_ref):
    y = xw_ref[...] * _dinv(degp_ref)
    y_ref[0] = y[:, :HID // 2]
    y_ref[1] = y[:, HID // 2:]


def _tc_scale1(xw, degp):
    grid = (NPAD // RB,)
    return pl.pallas_call(
        _scale1_body,
        grid=grid,
        in_specs=[
            pl.BlockSpec((RB, HID), lambda i: (i, 0)),
            pl.BlockSpec((NC, RB, DEG_W), lambda i: (0, i, 0)),
        ],
        out_specs=pl.BlockSpec((2, RB, HID // 2), lambda i: (0, i, 0)),
        out_shape=jax.ShapeDtypeStruct((2, NPAD, HID // 2), jnp.float32),
    )(xw, degp)


def _lin2_body(sp_ref, degp_ref, b1_ref, w2_ref, y2_ref):
    dinv = _dinv(degp_ref)
    tot = sp_ref[0] + sp_ref[1]
    pre = dinv * jnp.concatenate([tot[0], tot[1]], axis=1) + b1_ref[...]
    h = jnp.maximum(pre, 0.0)
    y2_ref[...] = jnp.dot(h, w2_ref[...],
                          preferred_element_type=jnp.float32) * dinv


def _tc_lin2(s1, degp, b1r, W2p):
    grid = (NPAD // RB,)
    return pl.pallas_call(
        _lin2_body,
        grid=grid,
        in_specs=[
            pl.BlockSpec((NC, 2, RB, HID // 2), lambda i: (0, 0, i, 0)),
            pl.BlockSpec((NC, RB, DEG_W), lambda i: (0, i, 0)),
            pl.BlockSpec((1, HID), lambda i: (0, 0)),
            pl.BlockSpec((HID, CPAD), lambda i: (0, 0)),
        ],
        out_specs=pl.BlockSpec((RB, CPAD), lambda i: (i, 0)),
        out_shape=jax.ShapeDtypeStruct((NPAD, CPAD), jnp.float32),
    )(s1, degp, b1r, W2p)


def _out_body(tp_ref, degp_ref, b2_ref, o_ref):
    dinv = _dinv(degp_ref)
    z = dinv * (tp_ref[0] + tp_ref[1]) + b2_ref[...]
    colid = lax.broadcasted_iota(jnp.int32, z.shape, 1)
    z = jnp.where(colid < CLS, z, -1e30)
    m = jnp.max(z, axis=1, keepdims=True)
    lse = jnp.log(jnp.sum(jnp.exp(z - m), axis=1, keepdims=True)) + m
    o_ref[...] = z - lse


def _tc_out(t1, degp, b2p):
    grid = (NPAD // RB,)
    return pl.pallas_call(
        _out_body,
        grid=grid,
        in_specs=[
            pl.BlockSpec((NC, RB, CPAD), lambda i: (0, i, 0)),
            pl.BlockSpec((NC, RB, DEG_W), lambda i: (0, i, 0)),
            pl.BlockSpec((1, CPAD), lambda i: (0, 0)),
        ],
        out_specs=pl.BlockSpec((RB, CPAD), lambda i: (i, 0)),
        out_shape=jax.ShapeDtypeStruct((NPAD, CPAD), jnp.float32),
    )(t1, degp, b2p)


def kernel(x, edge_index, W1, b1, W2, b2):
    ei = edge_index.astype(jnp.int32)
    padn = NW * EPT - E
    rowp = jnp.concatenate(
        [ei[0], jnp.full((padn,), N, jnp.int32)]).reshape(NW, NCHUNK, CHUNK)
    colp = jnp.concatenate(
        [ei[1], jnp.full((padn,), N, jnp.int32)]).reshape(NW, NCHUNK, CHUNK)
    xpad = jnp.zeros((NPAD, F_IN), jnp.float32).at[:N].set(x)
    ones16 = jnp.ones((DCHUNK, DEG_W), jnp.float32)
    z16 = jnp.zeros((NPAD, DEG_W), jnp.float32)
    z64 = jnp.zeros((NPAD, HID // 2), jnp.float32)
    z48 = jnp.zeros((NPAD, CPAD), jnp.float32)
    W2p = jnp.zeros((HID, CPAD), jnp.float32).at[:, :CLS].set(W2)
    b2p = jnp.zeros((1, CPAD), jnp.float32).at[0, :CLS].set(b2)
    b1r = b1.reshape(1, HID)

    degp = _sc_degree(colp.reshape(NW, DNCHUNK, DCHUNK), ones16, z16)
    xw = _tc_mm1(xpad, W1)
    y1 = _tc_scale1(xw, degp)
    s1 = _sc_propagate(y1, rowp, colp, z64, HID // 2, 2)
    y2 = _tc_lin2(s1, degp, b1r, W2p)
    t1 = _sc_propagate(y2.reshape(1, NPAD, CPAD), rowp, colp, z48, CPAD, 1)
    out = _tc_out(t1.reshape(NC, NPAD, CPAD), degp, b2p)
    return out[:N, :CLS]
